# baseline (speedup 1.0000x reference)
import jax
import jax.numpy as jnp
from jax import lax
from jax.experimental import pallas as pl
from jax.experimental.pallas import tpu as pltpu
from jax.experimental.pallas import tpu_sc as plsc

_NC = 2
_NS = 16
_NW = _NC * _NS
_L = 16
_B = 4096 * 50
_D = 128
_NQ = 257
_CHUNK = 128
_NBUF = 2
_PER_W = _B // _NW
_NCHUNK = _PER_W // _CHUNK


def _sc_body(x_hbm, q_hbm, w_hbm, out_hbm, qbuf, xall, wbuf, rows, wsem):
    wid = lax.axis_index("s") * _NC + lax.axis_index("c")
    base = wid * _PER_W

    pltpu.sync_copy(q_hbm, qbuf)
    pltpu.sync_copy(w_hbm, wbuf)
    pltpu.sync_copy(x_hbm.at[pl.ds(base, _PER_W)], xall)

    def fill_chunk(c, b):
        def group_block(g, carry):
            xv = xall[pl.ds(c * _CHUNK + g * _L, _L)]
            lo = jnp.zeros((_L,), jnp.int32)
            hi = jnp.full((_L,), 255, jnp.int32)
            for _ in range(8):
                mid = lax.shift_right_arithmetic(lo + hi, 1)
                edge = plsc.load_gather(qbuf, [mid + 1])
                go_right = edge < xv
                lo = jnp.where(go_right, mid + 1, lo)
                hi = jnp.where(go_right, hi, mid)
            ivec = lax.iota(jnp.int32, _L) + g * _L
            zero = jnp.zeros((_L,), jnp.int32)

            def dim_block(d0, inner):
                for j in range(16):
                    dv = zero + (d0 * 16 + j)
                    val = plsc.load_gather(wbuf, [lo, dv])
                    plsc.store_scatter(rows.at[b], [ivec, dv], val)
                return inner

            lax.fori_loop(0, _D // 16, dim_block, 0)
            return carry

        lax.fori_loop(0, _CHUNK // _L, group_block, 0)

    def start_write(c, b):
        pltpu.make_async_copy(
            rows.at[b], out_hbm.at[pl.ds(base + c * _CHUNK, _CHUNK)], wsem.at[b]
        ).start()

    def wait_write(c, b):
        pltpu.make_async_copy(
            rows.at[b], out_hbm.at[pl.ds(base + c * _CHUNK, _CHUNK)], wsem.at[b]
        ).wait()

    for b in range(_NBUF):
        fill_chunk(b, b)
        start_write(b, b)

    def ring_block(k, carry):
        c0 = k * _NBUF
        for b in range(_NBUF):
            c = c0 + b
            wait_write(c, b)
            fill_chunk(c + _NBUF, b)
            start_write(c + _NBUF, b)
        return carry

    lax.fori_loop(0, _NCHUNK // _NBUF - 1, ring_block, 0)

    c0 = _NCHUNK - _NBUF
    for b in range(_NBUF):
        wait_write(c0 + b, b)


@jax.jit
def _run(x_flat, quantiles, W):
    mesh = plsc.VectorSubcoreMesh(core_axis_name="c", subcore_axis_name="s")
    return pl.kernel(
        _sc_body,
        out_type=jax.ShapeDtypeStruct((_B, _D), jnp.float32),
        mesh=mesh,
        scratch_types=[
            pltpu.VMEM((_NQ,), jnp.float32),
            pltpu.VMEM((_PER_W,), jnp.float32),
            pltpu.VMEM((256, _D), jnp.float32),
            pltpu.VMEM((_NBUF, _CHUNK, _D), jnp.float32),
            pltpu.SemaphoreType.DMA((_NBUF,)),
        ],
        compiler_params=pltpu.CompilerParams(needs_layout_passes=False),
    )(x_flat, quantiles, W)


def kernel(x, quantiles, W):
    x_flat = x.reshape(_B)
    out = _run(x_flat, quantiles, W)
    return out.reshape(x.shape[0], x.shape[1], _D)


# speedup vs baseline: 1.9043x; 1.9043x over previous
import jax
import jax.numpy as jnp
from jax import lax
from jax.experimental import pallas as pl
from jax.experimental.pallas import tpu as pltpu
from jax.experimental.pallas import tpu_sc as plsc

_NC = 2
_NS = 16
_NW = _NC * _NS
_L = 16
_B = 4096 * 50
_D = 128
_NQ = 257
_CHUNK = 128
_NBUF = 2
_PER_W = _B // _NW
_NCHUNK = _PER_W // _CHUNK


def _sc_body(x_hbm, q_hbm, w_hbm, out_hbm, qbuf, xall, wbuf, rows0, rows1, wsem):
    rows = (rows0, rows1)
    wid = lax.axis_index("s") * _NC + lax.axis_index("c")
    base = wid * _PER_W

    pltpu.sync_copy(q_hbm, qbuf)
    pltpu.sync_copy(w_hbm, wbuf)
    pltpu.sync_copy(x_hbm.at[pl.ds(base, _PER_W)], xall)

    lane = lax.iota(jnp.int32, _L)
    lane128 = lane * _D

    def fill_chunk(c, b):
        rowsb = rows[b]

        @plsc.parallel_loop(0, _CHUNK // _L, unroll=2)
        def group_block(g):
            xv = xall[pl.ds(c * _CHUNK + g * _L, _L)]
            lo = jnp.zeros((_L,), jnp.int32)
            hi = jnp.full((_L,), 255, jnp.int32)
            for _ in range(8):
                mid = lax.shift_right_arithmetic(lo + hi, 1)
                edge = plsc.load_gather(qbuf, [mid + 1])
                go_right = edge < xv
                lo = jnp.where(go_right, mid + 1, lo)
                hi = jnp.where(go_right, hi, mid)
            src = lo * _D
            dst = lane128 + g * (_L * _D)

            @plsc.parallel_loop(0, _D, unroll=16)
            def dim_block(d):
                val = plsc.load_gather(wbuf, [src + d])
                plsc.store_scatter(rowsb, [dst + d], val)

    def start_write(c, b):
        pltpu.make_async_copy(
            rows[b],
            out_hbm.at[pl.ds((base + c * _CHUNK) * _D, _CHUNK * _D)],
            wsem.at[b],
        ).start()

    def wait_write(c, b):
        pltpu.make_async_copy(
            rows[b],
            out_hbm.at[pl.ds((base + c * _CHUNK) * _D, _CHUNK * _D)],
            wsem.at[b],
        ).wait()

    for b in range(_NBUF):
        fill_chunk(b, b)
        start_write(b, b)

    def ring_block(k, carry):
        c0 = k * _NBUF
        for b in range(_NBUF):
            c = c0 + b
            wait_write(c, b)
            fill_chunk(c + _NBUF, b)
            start_write(c + _NBUF, b)
        return carry

    lax.fori_loop(0, _NCHUNK // _NBUF - 1, ring_block, 0)

    c0 = _NCHUNK - _NBUF
    for b in range(_NBUF):
        wait_write(c0 + b, b)


@jax.jit
def _run(x_flat, q, w_flat):
    mesh = plsc.VectorSubcoreMesh(core_axis_name="c", subcore_axis_name="s")
    return pl.kernel(
        _sc_body,
        out_type=jax.ShapeDtypeStruct((_B * _D,), jnp.float32),
        mesh=mesh,
        scratch_types=[
            pltpu.VMEM((_NQ,), jnp.float32),
            pltpu.VMEM((_PER_W,), jnp.float32),
            pltpu.VMEM((256 * _D,), jnp.float32),
            pltpu.VMEM((_CHUNK * _D,), jnp.float32),
            pltpu.VMEM((_CHUNK * _D,), jnp.float32),
            pltpu.SemaphoreType.DMA((_NBUF,)),
        ],
        compiler_params=pltpu.CompilerParams(needs_layout_passes=False),
    )(x_flat, q, w_flat)


def kernel(x, quantiles, W):
    x_flat = x.reshape(_B)
    out = _run(x_flat, quantiles, W.reshape(-1))
    return out.reshape(x.shape[0], x.shape[1], _D)


# speedup vs baseline: 3.9736x; 2.0867x over previous
import jax
import jax.numpy as jnp
from jax import lax
from jax.experimental import pallas as pl
from jax.experimental.pallas import tpu as pltpu
from jax.experimental.pallas import tpu_sc as plsc

_NC = 2
_NS = 16
_NW = _NC * _NS
_L = 16
_B = 4096 * 50
_D = 128
_NQ = 257
_CHUNK = 128
_NBUF = 2
_PER_W = _B // _NW
_NCHUNK = _PER_W // _CHUNK


def _sc_body(x_hbm, q_hbm, w_hbm, out_hbm, qbuf, xall, wbuf, rows0, rows1, wsem):
    rows = (rows0, rows1)
    wid = lax.axis_index("s") * _NC + lax.axis_index("c")
    base = wid * _PER_W

    pltpu.sync_copy(q_hbm, qbuf)
    pltpu.sync_copy(w_hbm, wbuf)
    pltpu.sync_copy(x_hbm.at[pl.ds(base, _PER_W)], xall)

    lane = lax.iota(jnp.int32, _L)
    lane128 = lane * _D

    def fill_chunk(c, b):
        rowsb = rows[b]

        @plsc.parallel_loop(0, _CHUNK // _L, unroll=2)
        def group_block(g):
            xv = xall[pl.ds(c * _CHUNK + g * _L, _L)]
            lo = jnp.zeros((_L,), jnp.int32)
            hi = jnp.full((_L,), 255, jnp.int32)
            for _ in range(8):
                mid = lax.shift_right_arithmetic(lo + hi, 1)
                edge = plsc.load_gather(qbuf, [mid + 1])
                go_right = edge < xv
                lo = jnp.where(go_right, mid + 1, lo)
                hi = jnp.where(go_right, hi, mid)
            src = lo * _D
            dst = lane128 + g * (_L * _D)

            @plsc.parallel_loop(0, _D, unroll=16)
            def dim_block(d):
                dj = (lane + d) & (_D - 1)
                val = plsc.load_gather(wbuf, [src + dj])
                plsc.store_scatter(rowsb, [dst + dj], val)

    def start_write(c, b):
        pltpu.make_async_copy(
            rows[b],
            out_hbm.at[pl.ds((base + c * _CHUNK) * _D, _CHUNK * _D)],
            wsem.at[b],
        ).start()

    def wait_write(c, b):
        pltpu.make_async_copy(
            rows[b],
            out_hbm.at[pl.ds((base + c * _CHUNK) * _D, _CHUNK * _D)],
            wsem.at[b],
        ).wait()

    for b in range(_NBUF):
        fill_chunk(b, b)
        start_write(b, b)

    def ring_block(k, carry):
        c0 = k * _NBUF
        for b in range(_NBUF):
            c = c0 + b
            wait_write(c, b)
            fill_chunk(c + _NBUF, b)
            start_write(c + _NBUF, b)
        return carry

    lax.fori_loop(0, _NCHUNK // _NBUF - 1, ring_block, 0)

    c0 = _NCHUNK - _NBUF
    for b in range(_NBUF):
        wait_write(c0 + b, b)


@jax.jit
def _run(x_flat, q, w_flat):
    mesh = plsc.VectorSubcoreMesh(core_axis_name="c", subcore_axis_name="s")
    return pl.kernel(
        _sc_body,
        out_type=jax.ShapeDtypeStruct((_B * _D,), jnp.float32),
        mesh=mesh,
        scratch_types=[
            pltpu.VMEM((_NQ,), jnp.float32),
            pltpu.VMEM((_PER_W,), jnp.float32),
            pltpu.VMEM((256 * _D,), jnp.float32),
            pltpu.VMEM((_CHUNK * _D,), jnp.float32),
            pltpu.VMEM((_CHUNK * _D,), jnp.float32),
            pltpu.SemaphoreType.DMA((_NBUF,)),
        ],
        compiler_params=pltpu.CompilerParams(needs_layout_passes=False),
    )(x_flat, q, w_flat)


def kernel(x, quantiles, W):
    x_flat = x.reshape(_B)
    out = _run(x_flat, quantiles, W.reshape(-1))
    return out.reshape(x.shape[0], x.shape[1], _D)
